# Initial kernel scaffold; baseline (speedup 1.0000x reference)
import jax
import jax.numpy as jnp
from jax.experimental import pallas as pl


def kernel(X, given_table, pos_table, t2v_w, t2v_b, vt_W, vt_b):
    raise NotImplementedError("write your pallas kernel here")


# speedup vs baseline: 9.5323x; 9.5323x over previous
import functools

import jax
import jax.numpy as jnp
import numpy as np
from jax.experimental import pallas as pl
from jax.experimental.pallas import tpu as pltpu

_B = 128
_L = 1600
_D_TIME = 6
_K = 6
_D_MODEL = 128
_JW = 40


def _embed_body(x_ref, r_ref, wf_ref, bf_ref, vtw_ref, bias_ref, pos_ref,
                o_ref):
    xb = x_ref[0]
    y_raw = xb[0:1, :]
    nanmask = jnp.isnan(y_raw)
    y = jnp.where(nanmask, 0.0, y_raw)
    x6 = xb[2:8, :]
    x6 = jnp.where(jnp.isnan(x6), 0.0, x6)

    xr = jnp.dot(r_ref[...], x6, preferred_element_type=jnp.float32)
    affine = xr * wf_ref[...] + bf_ref[...]

    j = jax.lax.broadcasted_iota(jnp.int32, (_JW, 1), 0)
    sin_rows = jnp.logical_and(j % _K >= 1, j < 36)
    v = jnp.where(sin_rows, jnp.sin(affine), affine)
    v = jnp.where(j == 36, y, v)
    v = jnp.where(j == 37, nanmask.astype(jnp.float32), v)

    out = jax.lax.dot_general(v, vtw_ref[...],
                              dimension_numbers=(((0,), (0,)), ((), ())),
                              preferred_element_type=jnp.float32)
    o_ref[0] = out + bias_ref[...] + pos_ref[...]


@jax.jit
def kernel(X, given_table, pos_table, t2v_w, t2v_b, vt_W, vt_b):
    B, _, L = X.shape
    d_model = pos_table.shape[1]

    wf = jnp.concatenate([t2v_w.reshape(-1),
                          jnp.zeros((4,), jnp.float32)]).reshape(_JW, 1)
    bf = jnp.concatenate([t2v_b.reshape(-1),
                          jnp.zeros((4,), jnp.float32)]).reshape(_JW, 1)
    vtw = jnp.concatenate([
        vt_W,
        (given_table[0] - given_table[1])[None, :],
        jnp.zeros((2, d_model), jnp.float32),
    ], axis=0)
    bias = (vt_b + given_table[1]).reshape(1, d_model)
    r = jnp.asarray(np.eye(_D_TIME, dtype=np.float32)
                    .repeat(_K, axis=0))
    r = jnp.concatenate([r, jnp.zeros((4, _D_TIME), jnp.float32)], axis=0)

    grid = (B,)
    out = pl.pallas_call(
        _embed_body,
        grid=grid,
        in_specs=[
            pl.BlockSpec((1, 8, L), lambda b: (b, 0, 0)),
            pl.BlockSpec((_JW, _D_TIME), lambda b: (0, 0)),
            pl.BlockSpec((_JW, 1), lambda b: (0, 0)),
            pl.BlockSpec((_JW, 1), lambda b: (0, 0)),
            pl.BlockSpec((_JW, d_model), lambda b: (0, 0)),
            pl.BlockSpec((1, d_model), lambda b: (0, 0)),
            pl.BlockSpec((L, d_model), lambda b: (0, 0)),
        ],
        out_specs=pl.BlockSpec((1, L, d_model), lambda b: (b, 0, 0)),
        out_shape=jax.ShapeDtypeStruct((B, L, d_model), jnp.float32),
    )(X, r, wf, bf, vtw, bias, pos_table)
    return out


# speedup vs baseline: 10.5596x; 1.1078x over previous
import functools

import jax
import jax.numpy as jnp
import numpy as np
from jax.experimental import pallas as pl
from jax.experimental.pallas import tpu as pltpu

_B = 128
_L = 1600
_D_TIME = 6
_K = 6
_D_MODEL = 128
_JW = 40


def _embed_body(x_ref, r_ref, wf_ref, bf_ref, vtw_ref, bias_ref, pos_ref,
                o_ref):
    xb = x_ref[0]
    y_raw = xb[0:1, :]
    nanmask = jnp.isnan(y_raw)
    y = jnp.where(nanmask, 0.0, y_raw)
    x6 = xb[2:8, :]
    x6 = jnp.where(jnp.isnan(x6), 0.0, x6)

    xr = jnp.dot(r_ref[...], x6, preferred_element_type=jnp.float32)
    affine = xr * wf_ref[...] + bf_ref[...]

    top = jnp.sin(affine[0:32, :])
    bot = affine[32:40, :]
    i = jax.lax.broadcasted_iota(jnp.int32, (8, 1), 0)
    bot = jnp.where(i == 6, y, bot)
    bot = jnp.where(i == 7, nanmask.astype(jnp.float32), bot)
    v = jnp.concatenate([top, bot], axis=0)

    out = jax.lax.dot_general(v, vtw_ref[...],
                              dimension_numbers=(((0,), (0,)), ((), ())),
                              preferred_element_type=jnp.float32)
    o_ref[0] = out + bias_ref[...] + pos_ref[...]


@jax.jit
def kernel(X, given_table, pos_table, t2v_w, t2v_b, vt_W, vt_b):
    B, _, L = X.shape
    d_model = pos_table.shape[1]
    f32 = jnp.float32

    z2 = jnp.zeros((2,), f32)
    wf = jnp.concatenate([t2v_w[:, 1:].reshape(-1), z2,
                          t2v_w[:, 0], z2]).reshape(_JW, 1)
    bf = jnp.concatenate([t2v_b[:, 1:].reshape(-1), z2,
                          t2v_b[:, 0], z2]).reshape(_JW, 1)
    rr = np.arange(30)
    perm_top = (rr // 5) * 6 + (rr % 5 + 1)
    perm_bot = np.arange(6) * 6
    vtw = jnp.concatenate([
        vt_W[perm_top],
        jnp.zeros((2, d_model), f32),
        vt_W[perm_bot],
        vt_W[36][None, :],
        (given_table[0] - given_table[1])[None, :],
    ], axis=0)
    bias = (vt_b + given_table[1]).reshape(1, d_model)

    r_np = np.zeros((_JW, _D_TIME), np.float32)
    r_np[np.arange(30), np.arange(30) // 5] = 1.0
    r_np[np.arange(32, 38), np.arange(6)] = 1.0
    r = jnp.asarray(r_np)

    grid = (B,)
    out = pl.pallas_call(
        _embed_body,
        grid=grid,
        in_specs=[
            pl.BlockSpec((1, 8, L), lambda b: (b, 0, 0)),
            pl.BlockSpec((_JW, _D_TIME), lambda b: (0, 0)),
            pl.BlockSpec((_JW, 1), lambda b: (0, 0)),
            pl.BlockSpec((_JW, 1), lambda b: (0, 0)),
            pl.BlockSpec((_JW, d_model), lambda b: (0, 0)),
            pl.BlockSpec((1, d_model), lambda b: (0, 0)),
            pl.BlockSpec((L, d_model), lambda b: (0, 0)),
        ],
        out_specs=pl.BlockSpec((1, L, d_model), lambda b: (b, 0, 0)),
        out_shape=jax.ShapeDtypeStruct((B, L, d_model), jnp.float32),
    )(X, r, wf, bf, vtw, bias, pos_table)
    return out


# speedup vs baseline: 13.0749x; 1.2382x over previous
import functools

import jax
import jax.numpy as jnp
import numpy as np
from jax.experimental import pallas as pl
from jax.experimental.pallas import tpu as pltpu

_B = 128
_L = 1600
_D_TIME = 6
_K = 6
_D_MODEL = 128
_JW = 40


def _embed_body(x_ref, r_ref, wf_ref, bf_ref, vtw_ref, bias_ref, pos_ref,
                o_ref):
    xb = x_ref[0]
    y_raw = xb[0:1, :]
    nanmask = jnp.isnan(y_raw)
    y = jnp.where(nanmask, 0.0, y_raw)
    x6 = xb[2:8, :]
    x6 = jnp.where(jnp.isnan(x6), 0.0, x6)

    xr = jnp.dot(r_ref[...], x6, preferred_element_type=jnp.float32)
    affine = xr * wf_ref[...] + bf_ref[...]

    t = affine[0:32, :]
    n_f = jnp.floor(t * 0.3183098861837907 + 0.5)
    parity = (n_f.astype(jnp.int32) & 1) << 31
    r = t - n_f * 3.140625
    r = r - n_f * 9.67653589793e-4
    s = r * r
    poly = 1.0 + s * (-0.16666650772094727 + s * (0.008332963101565838
                      + s * (-0.00019804720068350434 + s * 2.5980341433751164e-06)))
    val = r * poly
    top = jax.lax.bitcast_convert_type(
        jax.lax.bitcast_convert_type(val, jnp.int32) ^ parity, jnp.float32)
    bot = affine[32:40, :]
    i = jax.lax.broadcasted_iota(jnp.int32, (8, 1), 0)
    bot = jnp.where(i == 6, y, bot)
    bot = jnp.where(i == 7, nanmask.astype(jnp.float32), bot)
    v = jnp.concatenate([top, bot], axis=0)

    out = jax.lax.dot_general(v, vtw_ref[...],
                              dimension_numbers=(((0,), (0,)), ((), ())),
                              preferred_element_type=jnp.float32)
    o_ref[0] = out + bias_ref[...] + pos_ref[...]


@jax.jit
def kernel(X, given_table, pos_table, t2v_w, t2v_b, vt_W, vt_b):
    B, _, L = X.shape
    d_model = pos_table.shape[1]
    f32 = jnp.float32

    z2 = jnp.zeros((2,), f32)
    wf = jnp.concatenate([t2v_w[:, 1:].reshape(-1), z2,
                          t2v_w[:, 0], z2]).reshape(_JW, 1)
    bf = jnp.concatenate([t2v_b[:, 1:].reshape(-1), z2,
                          t2v_b[:, 0], z2]).reshape(_JW, 1)
    rr = np.arange(30)
    perm_top = (rr // 5) * 6 + (rr % 5 + 1)
    perm_bot = np.arange(6) * 6
    vtw = jnp.concatenate([
        vt_W[perm_top],
        jnp.zeros((2, d_model), f32),
        vt_W[perm_bot],
        vt_W[36][None, :],
        (given_table[0] - given_table[1])[None, :],
    ], axis=0)
    bias = (vt_b + given_table[1]).reshape(1, d_model)

    r_np = np.zeros((_JW, _D_TIME), np.float32)
    r_np[np.arange(30), np.arange(30) // 5] = 1.0
    r_np[np.arange(32, 38), np.arange(6)] = 1.0
    r = jnp.asarray(r_np)

    grid = (B,)
    out = pl.pallas_call(
        _embed_body,
        grid=grid,
        in_specs=[
            pl.BlockSpec((1, 8, L), lambda b: (b, 0, 0)),
            pl.BlockSpec((_JW, _D_TIME), lambda b: (0, 0)),
            pl.BlockSpec((_JW, 1), lambda b: (0, 0)),
            pl.BlockSpec((_JW, 1), lambda b: (0, 0)),
            pl.BlockSpec((_JW, d_model), lambda b: (0, 0)),
            pl.BlockSpec((1, d_model), lambda b: (0, 0)),
            pl.BlockSpec((L, d_model), lambda b: (0, 0)),
        ],
        out_specs=pl.BlockSpec((1, L, d_model), lambda b: (b, 0, 0)),
        out_shape=jax.ShapeDtypeStruct((B, L, d_model), jnp.float32),
    )(X, r, wf, bf, vtw, bias, pos_table)
    return out
